# baseline (speedup 1.0000x reference)
import functools

import jax
import jax.numpy as jnp
from jax import lax
from jax.experimental import pallas as pl
from jax.experimental.pallas import tpu as pltpu
from jax.experimental.pallas import tpu_sc as plsc

N = 10000
E = 320000
D = 128

NC, NS, L = 2, 16, 16
EPT = E // NS
W = 80
EB = 800
NBATCH = EPT // EB
WPB = EB // W
CH = 80
NCHUNK = N // CH
NREG = D // L


def _mm_body(x_ref, w_ref, o_ref):
    o_ref[...] = jnp.dot(x_ref[...], w_ref[...],
                         preferred_element_type=jnp.float32)


def _matmul(x, w):
    return pl.pallas_call(
        _mm_body,
        grid=(10,),
        in_specs=[pl.BlockSpec((1000, D), lambda i: (i, 0)),
                  pl.BlockSpec((D, D), lambda i: (0, 0))],
        out_specs=pl.BlockSpec((1000, D), lambda i: (i, 0)),
        out_shape=jax.ShapeDtypeStruct((N, D), jnp.float32),
    )(x, w)


@functools.partial(
    pl.kernel,
    out_type=(jax.ShapeDtypeStruct((N, D), jnp.float32),
              jax.ShapeDtypeStruct((N, D), jnp.float32)),
    mesh=plsc.VectorSubcoreMesh(core_axis_name="c", subcore_axis_name="s"),
    scratch_types=[
        pltpu.VMEM_SHARED((N, D), jnp.float32),
        pltpu.VMEM((EB,), jnp.int32),
        pltpu.VMEM((EB,), jnp.int32),
        pltpu.VMEM((EB,), jnp.float32),
        pltpu.VMEM((W,), jnp.int32),
        pltpu.VMEM((W,), jnp.int32),
        pltpu.VMEM((W, D), jnp.float32),
        pltpu.VMEM((W, D), jnp.float32),
        pltpu.VMEM((W, D), jnp.float32),
        pltpu.VMEM((W, D), jnp.float32),
        pltpu.VMEM((D,), jnp.float32),
        pltpu.SemaphoreType.DMA,
        pltpu.SemaphoreType.DMA,
        pltpu.SemaphoreType.DMA,
        pltpu.SemaphoreType.DMA,
    ],
)
def _sc_agg(h1, h2, src1, dst1, src2, dst2, vals1, vals2, b1, b2, out1, out2,
            acc, srcb, dstb, valb, dstw0, dstw1, pbuf0, pbuf1, fbuf0, fbuf1,
            bias_v, gsem0, gsem1, ssem0, ssem1):
    c = lax.axis_index("c")
    s = lax.axis_index("s")

    def run_graph(h, src, dst, vals, bias, out):
        obuf = pbuf0
        def zrow(r, carry):
            for j in range(NREG):
                obuf[r, pl.ds(j * L, L)] = jnp.zeros((L,), jnp.float32)
            return carry
        lax.fori_loop(0, CH, zrow, 0)
        nch_mine = (NCHUNK - s + NS - 1) // NS

        def zcp(k, carry):
            r0 = (s + k * NS) * CH
            pltpu.sync_copy(obuf, acc.at[pl.ds(r0, CH), :])
            return carry
        lax.fori_loop(0, nch_mine, zcp, 0)
        plsc.subcore_barrier()

        def scale(pbuf, fbuf, dw, ebase):
            def sc16(g, carry):
                dw[pl.ds(g * L, L)] = dstb[pl.ds(ebase + g * L, L)]
                vblk = valb[pl.ds(ebase + g * L, L)]
                for l in range(L):
                    vs = jnp.full((L,), vblk[l], jnp.float32)
                    r = g * L + l
                    for j in range(NREG):
                        fbuf[r, pl.ds(j * L, L)] = (
                            pbuf[r, pl.ds(j * L, L)] * vs)
                return carry
            lax.fori_loop(0, W // L, sc16, 0)

        pbufs = (pbuf0, pbuf1)
        fbufs = (fbuf0, fbuf1)
        dstws = (dstw0, dstw1)
        gsems = (gsem0, gsem1)
        ssems = (ssem0, ssem1)
        e0 = s * EPT

        def batch(b, carry):
            eb0 = e0 + b * EB
            pltpu.sync_copy(src.at[pl.ds(eb0, EB)], srcb)
            pltpu.sync_copy(dst.at[pl.ds(eb0, EB)], dstb)
            pltpu.sync_copy(vals.at[pl.ds(eb0, EB)], valb)

            def gissue(w):
                k = w % 2
                return pltpu.async_copy(
                    h.at[srcb.at[pl.ds(w * W, W)]], pbufs[k], gsems[k])

            gds = {0: gissue(0), 1: gissue(1)}
            sds = {}
            for w in range(WPB):
                k = w % 2
                if w >= 2:
                    sds[w - 2].wait()
                gds[w].wait()
                scale(pbufs[k], fbufs[k], dstws[k], w * W)
                if w + 2 < WPB:
                    gds[w + 2] = gissue(w + 2)
                sds[w] = pltpu.async_copy(
                    fbufs[k], acc.at[dstws[k]], ssems[k], add=True)
            sds[WPB - 2].wait()
            sds[WPB - 1].wait()
            return carry
        lax.fori_loop(0, NBATCH, batch, 0)
        plsc.subcore_barrier()

        pltpu.sync_copy(bias, bias_v)

        def ep(k, carry):
            r0 = (s + k * NS) * CH
            pltpu.sync_copy(acc.at[pl.ds(r0, CH), :], obuf)

            def rw(r, carry2):
                for j in range(NREG):
                    x = obuf[r, pl.ds(j * L, L)] + bias_v[pl.ds(j * L, L)]
                    obuf[r, pl.ds(j * L, L)] = jnp.maximum(x, 0.0)
                return carry2
            lax.fori_loop(0, CH, rw, 0)
            pltpu.sync_copy(obuf, out.at[pl.ds(r0, CH), :])
            return carry
        lax.fori_loop(0, nch_mine, ep, 0)

    @pl.when(c == 0)
    def _():
        run_graph(h1, src1, dst1, vals1, b1, out1)

    @pl.when(c == 1)
    def _():
        run_graph(h2, src2, dst2, vals2, b2, out2)


def kernel(x1, x2, out_edge_index, in_edge_index, out_vals, in_vals,
           out_weight, in_weight, bias1, bias2):
    h1 = _matmul(x1, out_weight)
    h2 = _matmul(x2, in_weight)
    return _sc_agg(h1, h2, out_edge_index[1], out_edge_index[0],
                   in_edge_index[1], in_edge_index[0],
                   out_vals, in_vals, bias1, bias2)


# speedup vs baseline: 1.0578x; 1.0578x over previous
import functools

import jax
import jax.numpy as jnp
from jax import lax
from jax.experimental import pallas as pl
from jax.experimental.pallas import tpu as pltpu
from jax.experimental.pallas import tpu_sc as plsc

N = 10000
E = 320000
D = 128

NC, NS, L = 2, 16, 16
EPT = E // NS
W = 80
EB = 2000
NBATCH = EPT // EB
WPB = EB // W
NPAIR = (WPB - 1) // 2
CH = 16
NCHUNK = N // CH
NREG = D // L


def _mm_body(x_ref, w_ref, o_ref):
    o_ref[...] = jnp.dot(x_ref[...], w_ref[...],
                         preferred_element_type=jnp.float32)


def _matmul(x, w):
    return pl.pallas_call(
        _mm_body,
        grid=(10,),
        in_specs=[pl.BlockSpec((1000, D), lambda i: (i, 0)),
                  pl.BlockSpec((D, D), lambda i: (0, 0))],
        out_specs=pl.BlockSpec((1000, D), lambda i: (i, 0)),
        out_shape=jax.ShapeDtypeStruct((N, D), jnp.float32),
    )(x, w)


@functools.partial(
    pl.kernel,
    out_type=(jax.ShapeDtypeStruct((N, D), jnp.float32),
              jax.ShapeDtypeStruct((N, D), jnp.float32)),
    mesh=plsc.VectorSubcoreMesh(core_axis_name="c", subcore_axis_name="s"),
    scratch_types=[
        pltpu.VMEM_SHARED((N, D), jnp.float32),
        pltpu.VMEM((EB,), jnp.int32),
        pltpu.VMEM((EB,), jnp.int32),
        pltpu.VMEM((EB,), jnp.float32),
        pltpu.VMEM((W,), jnp.int32),
        pltpu.VMEM((W,), jnp.int32),
        pltpu.VMEM((W,), jnp.int32),
        pltpu.VMEM((W,), jnp.int32),
        pltpu.VMEM((W, D), jnp.float32),
        pltpu.VMEM((W, D), jnp.float32),
        pltpu.VMEM((W, D), jnp.float32),
        pltpu.VMEM((W, D), jnp.float32),
        pltpu.VMEM((CH, D), jnp.float32),
        pltpu.VMEM((D,), jnp.float32),
        pltpu.SemaphoreType.DMA,
        pltpu.SemaphoreType.DMA,
        pltpu.SemaphoreType.DMA,
        pltpu.SemaphoreType.DMA,
        pltpu.SemaphoreType.DMA,
        pltpu.SemaphoreType.DMA,
        pltpu.SemaphoreType.DMA,
        pltpu.SemaphoreType.DMA,
    ],
)
def _sc_agg(h1, h2, src1, dst1, src2, dst2, vals1, vals2, b1, b2, out1, out2,
            acc, srcb, dstb, valb, dstw0, dstw1, dstw2, dstw3,
            rows0, rows1, rows2, rows3, obuf, bias_v,
            gsem0, gsem1, gsem2, gsem3, ssem0, ssem1, ssem2, ssem3):
    c = lax.axis_index("c")
    s = lax.axis_index("s")

    def run_graph(h, src, dst, vals, bias, out):
        def zrow(r, carry):
            for j in range(NREG):
                obuf[r, pl.ds(j * L, L)] = jnp.zeros((L,), jnp.float32)
            return carry
        lax.fori_loop(0, CH, zrow, 0)
        nch_mine = (NCHUNK - s + NS - 1) // NS

        def zcp(k, carry):
            r0 = (s + k * NS) * CH
            pltpu.sync_copy(obuf, acc.at[pl.ds(r0, CH), :])
            return carry
        lax.fori_loop(0, nch_mine, zcp, 0)
        plsc.subcore_barrier()

        def scale(rbuf, dw, ebase):
            def sc16(g, carry):
                dw[pl.ds(g * L, L)] = dstb[pl.ds(ebase + g * L, L)]
                vblk = valb[pl.ds(ebase + g * L, L)]
                for l in range(L):
                    vs = jnp.full((L,), vblk[l], jnp.float32)
                    r = g * L + l
                    for j in range(NREG):
                        rbuf[r, pl.ds(j * L, L)] = (
                            rbuf[r, pl.ds(j * L, L)] * vs)
                return carry
            lax.fori_loop(0, W // L, sc16, 0)

        rows = (rows0, rows1, rows2, rows3)
        dstws = (dstw0, dstw1, dstw2, dstw3)
        gsems = (gsem0, gsem1, gsem2, gsem3)
        ssems = (ssem0, ssem1, ssem2, ssem3)
        e0 = s * EPT

        def batch(b, carry):
            eb0 = e0 + b * EB
            pltpu.sync_copy(src.at[pl.ds(eb0, EB)], srcb)
            pltpu.sync_copy(dst.at[pl.ds(eb0, EB)], dstb)
            pltpu.sync_copy(vals.at[pl.ds(eb0, EB)], valb)

            def gissue(w):
                k = w % 4
                return pltpu.async_copy(
                    h.at[srcb.at[pl.ds(w * W, W)]], rows[k], gsems[k])

            gds = {0: gissue(0), 1: gissue(1), 2: gissue(2)}
            sds = {}
            for w in range(WPB):
                k = w % 4
                if w >= 1:
                    sds[w - 1].wait()
                if w + 3 < WPB:
                    gds[w + 3] = gissue(w + 3)
                gds[w].wait()
                scale(rows[k], dstws[k], w * W)
                sds[w] = pltpu.async_copy(
                    rows[k], acc.at[dstws[k]], ssems[k], add=True)
            sds[WPB - 1].wait()
            return carry
        lax.fori_loop(0, NBATCH, batch, 0)
        plsc.subcore_barrier()

        pltpu.sync_copy(bias, bias_v)

        def ep(k, carry):
            r0 = (s + k * NS) * CH
            pltpu.sync_copy(acc.at[pl.ds(r0, CH), :], obuf)

            def rw(r, carry2):
                for j in range(NREG):
                    x = obuf[r, pl.ds(j * L, L)] + bias_v[pl.ds(j * L, L)]
                    obuf[r, pl.ds(j * L, L)] = jnp.maximum(x, 0.0)
                return carry2
            lax.fori_loop(0, CH, rw, 0)
            pltpu.sync_copy(obuf, out.at[pl.ds(r0, CH), :])
            return carry
        lax.fori_loop(0, nch_mine, ep, 0)

    @pl.when(c == 0)
    def _():
        run_graph(h1, src1, dst1, vals1, b1, out1)

    @pl.when(c == 1)
    def _():
        run_graph(h2, src2, dst2, vals2, b2, out2)


def kernel(x1, x2, out_edge_index, in_edge_index, out_vals, in_vals,
           out_weight, in_weight, bias1, bias2):
    h1 = _matmul(x1, out_weight)
    h2 = _matmul(x2, in_weight)
    return _sc_agg(h1, h2, out_edge_index[1], out_edge_index[0],
                   in_edge_index[1], in_edge_index[0],
                   out_vals, in_vals, bias1, bias2)


# speedup vs baseline: 1.2247x; 1.1577x over previous
import functools

import jax
import jax.numpy as jnp
from jax import lax
from jax.experimental import pallas as pl
from jax.experimental.pallas import tpu as pltpu
from jax.experimental.pallas import tpu_sc as plsc

N = 10000
E = 320000
D = 128

NC, NS, L = 2, 16, 16
EPT = E // NS
W = 80
EB = 2000
NBATCH = EPT // EB
WPB = EB // W
NPAIR = (WPB - 1) // 2
CH = 80
NCHUNK = N // CH
NREG = D // L


def _mm_body(x_ref, w_ref, o_ref):
    o_ref[...] = jnp.dot(x_ref[...], w_ref[...],
                         preferred_element_type=jnp.float32)


def _matmul(x, w):
    return pl.pallas_call(
        _mm_body,
        grid=(10,),
        in_specs=[pl.BlockSpec((1000, D), lambda i: (i, 0)),
                  pl.BlockSpec((D, D), lambda i: (0, 0))],
        out_specs=pl.BlockSpec((1000, D), lambda i: (i, 0)),
        out_shape=jax.ShapeDtypeStruct((N, D), jnp.float32),
    )(x, w)


@functools.partial(
    pl.kernel,
    out_type=(jax.ShapeDtypeStruct((N, D), jnp.float32),
              jax.ShapeDtypeStruct((N, D), jnp.float32)),
    mesh=plsc.VectorSubcoreMesh(core_axis_name="c", subcore_axis_name="s"),
    scratch_types=[
        pltpu.VMEM_SHARED((N, D), jnp.float32),
        pltpu.VMEM((EB,), jnp.int32),
        pltpu.VMEM((EB,), jnp.int32),
        pltpu.VMEM((EB,), jnp.float32),
        pltpu.VMEM((W,), jnp.int32),
        pltpu.VMEM((W,), jnp.int32),
        pltpu.VMEM((W,), jnp.int32),
        pltpu.VMEM((W,), jnp.int32),
        pltpu.VMEM((W, D), jnp.float32),
        pltpu.VMEM((W, D), jnp.float32),
        pltpu.VMEM((W, D), jnp.float32),
        pltpu.VMEM((W, D), jnp.float32),
        pltpu.VMEM((D,), jnp.float32),
        pltpu.SemaphoreType.DMA,
        pltpu.SemaphoreType.DMA,
        pltpu.SemaphoreType.DMA,
        pltpu.SemaphoreType.DMA,
        pltpu.SemaphoreType.DMA,
        pltpu.SemaphoreType.DMA,
        pltpu.SemaphoreType.DMA,
        pltpu.SemaphoreType.DMA,
    ],
)
def _sc_agg(h1, h2, src1, dst1, src2, dst2, vals1, vals2, b1, b2, out1, out2,
            acc, srcb, dstb, valb, dstw0, dstw1, dstw2, dstw3,
            rows0, rows1, rows2, rows3, bias_v,
            gsem0, gsem1, gsem2, gsem3, ssem0, ssem1, ssem2, ssem3):
    c = lax.axis_index("c")
    s = lax.axis_index("s")

    def run_graph(h, src, dst, vals, bias, out):
        obuf = rows0
        def zrow(r, carry):
            for j in range(NREG):
                obuf[r, pl.ds(j * L, L)] = jnp.zeros((L,), jnp.float32)
            return carry
        lax.fori_loop(0, CH, zrow, 0)
        nch_mine = (NCHUNK - s + NS - 1) // NS

        def zcp(k, carry):
            r0 = (s + k * NS) * CH
            pltpu.sync_copy(obuf, acc.at[pl.ds(r0, CH), :])
            return carry
        lax.fori_loop(0, nch_mine, zcp, 0)
        plsc.subcore_barrier()

        def scale(rbuf, dw, ebase):
            def sc16(g, carry):
                dw[pl.ds(g * L, L)] = dstb[pl.ds(ebase + g * L, L)]
                vblk = valb[pl.ds(ebase + g * L, L)]
                for l in range(L):
                    vs = jnp.full((L,), vblk[l], jnp.float32)
                    r = g * L + l
                    for j in range(NREG):
                        rbuf[r, pl.ds(j * L, L)] = (
                            rbuf[r, pl.ds(j * L, L)] * vs)
                return carry
            lax.fori_loop(0, W // L, sc16, 0)

        rows = (rows0, rows1, rows2, rows3)
        dstws = (dstw0, dstw1, dstw2, dstw3)
        gsems = (gsem0, gsem1, gsem2, gsem3)
        ssems = (ssem0, ssem1, ssem2, ssem3)
        e0 = s * EPT

        def batch(b, carry):
            eb0 = e0 + b * EB
            i1 = pltpu.async_copy(src.at[pl.ds(eb0, EB)], srcb, ssem0)
            i2 = pltpu.async_copy(dst.at[pl.ds(eb0, EB)], dstb, ssem1)
            i3 = pltpu.async_copy(vals.at[pl.ds(eb0, EB)], valb, ssem2)
            i1.wait()
            i2.wait()
            i3.wait()

            def gissue(w):
                k = w % 4
                return pltpu.async_copy(
                    h.at[srcb.at[pl.ds(w * W, W)]], rows[k], gsems[k])

            gds = {0: gissue(0), 1: gissue(1)}
            sds = {}
            for w in range(WPB):
                k = w % 4
                if w >= 2:
                    sds[w - 2].wait()
                if w + 2 < WPB:
                    gds[w + 2] = gissue(w + 2)
                gds[w].wait()
                scale(rows[k], dstws[k], w * W)
                sds[w] = pltpu.async_copy(
                    rows[k], acc.at[dstws[k]], ssems[k], add=True)
            sds[WPB - 2].wait()
            sds[WPB - 1].wait()
            return carry
        lax.fori_loop(0, NBATCH, batch, 0)
        plsc.subcore_barrier()

        pltpu.sync_copy(bias, bias_v)

        def ep(k, carry):
            r0 = (s + k * NS) * CH
            pltpu.sync_copy(acc.at[pl.ds(r0, CH), :], obuf)

            def rw(r, carry2):
                for j in range(NREG):
                    x = obuf[r, pl.ds(j * L, L)] + bias_v[pl.ds(j * L, L)]
                    obuf[r, pl.ds(j * L, L)] = jnp.maximum(x, 0.0)
                return carry2
            lax.fori_loop(0, CH, rw, 0)
            pltpu.sync_copy(obuf, out.at[pl.ds(r0, CH), :])
            return carry
        lax.fori_loop(0, nch_mine, ep, 0)

    @pl.when(c == 0)
    def _():
        run_graph(h1, src1, dst1, vals1, b1, out1)

    @pl.when(c == 1)
    def _():
        run_graph(h2, src2, dst2, vals2, b2, out2)


def kernel(x1, x2, out_edge_index, in_edge_index, out_vals, in_vals,
           out_weight, in_weight, bias1, bias2):
    h1 = _matmul(x1, out_weight)
    h2 = _matmul(x2, in_weight)
    return _sc_agg(h1, h2, out_edge_index[1], out_edge_index[0],
                   in_edge_index[1], in_edge_index[0],
                   out_vals, in_vals, bias1, bias2)
